# baseline (speedup 1.0000x reference)
import jax
import jax.numpy as jnp
from jax import lax
from jax.experimental import pallas as pl
from jax.experimental.pallas import tpu as pltpu

B = 8192
D = 256
BM = 256
BN = 512
NCORE = 2
NJ = (B // BM) // NCORE
NT = B // BN
INV_SQRT_T = 3.7796447300922722


def _loss_kernel(featsT_ref, comb_ref, labcol_ref, out_s_ref, out_c_ref,
                 scf_ref):
    c = pl.program_id(0)
    j = pl.program_id(1)
    r = c * NJ + j

    @pl.when(j == 0)
    def _prologue():
        ft = featsT_ref[...]
        ss = jnp.sum(ft * ft, axis=0, keepdims=True)
        inv = lax.rsqrt(ss) * INV_SQRT_T
        scf_ref[...] = ft * inv

    lhs = scf_ref[:, pl.ds(pl.multiple_of(r * BM, BM), BM)]
    rl = labcol_ref[...][:, 0:1]
    rid = lax.broadcasted_iota(jnp.int32, (BM, 1), 0) + r * BM

    acc_e = jnp.zeros((BM, 128), jnp.float32)
    acc_p = jnp.zeros((BM, 128), jnp.float32)
    acc_c = jnp.zeros((BM, 128), jnp.float32)

    def fold(x):
        return (x[:, 0:128] + x[:, 128:256]) + (x[:, 256:384] + x[:, 384:512])

    for jc in range(NT):
        rhs = scf_ref[:, jc * BN:(jc + 1) * BN]
        s = lax.dot_general(lhs, rhs, (((0,), (0,)), ((), ())),
                            preferred_element_type=jnp.float32,
                            precision=lax.Precision.HIGHEST)
        ct = comb_ref[0:1, jc * BN:(jc + 1) * BN]
        cid = lax.broadcasted_iota(jnp.int32, (1, BN), 1) + jc * BN
        eq = rl == ct
        dne = rid != cid
        pos = jnp.logical_and(eq, dne)
        e = jnp.where(dne, jnp.exp(s), 0.0)
        ps = jnp.where(pos, s, 0.0)
        cs = jnp.where(pos, 1.0, 0.0)
        acc_e = acc_e + fold(e)
        acc_p = acc_p + fold(ps)
        acc_c = acc_c + fold(cs)

    se = jnp.sum(acc_e, axis=1, keepdims=True)
    lse = jnp.log(se)
    cnt = jnp.sum(acc_c, axis=1, keepdims=True)
    psum = jnp.sum(acc_p, axis=1, keepdims=True)
    mean = (psum - cnt * lse) / (cnt + 1e-9)
    valid = cnt > 0
    contrib = jnp.where(valid, mean, 0.0)
    nv = jnp.where(valid, 1.0, 0.0)
    srow = jnp.sum(contrib, axis=0, keepdims=True)
    nrow = jnp.sum(nv, axis=0, keepdims=True)
    out_s_ref[...] = jnp.broadcast_to(srow, (1, 128)).reshape(1, 1, 128)
    out_c_ref[...] = jnp.broadcast_to(nrow, (1, 128)).reshape(1, 1, 128)


def kernel(features, concept_labels, class_labels):
    featsT = features.T
    comb = (concept_labels.astype(jnp.int32) * 16
            + class_labels.astype(jnp.int32))
    comb_row = comb.reshape(1, B)
    comb_col = jnp.broadcast_to(comb.reshape(B, 1), (B, 128))

    nblk = NCORE * NJ
    out_s, out_c = pl.pallas_call(
        _loss_kernel,
        grid=(NCORE, NJ),
        in_specs=[
            pl.BlockSpec((D, B), lambda c, j: (0, 0)),
            pl.BlockSpec((1, B), lambda c, j: (0, 0)),
            pl.BlockSpec((BM, 128), lambda c, j: (c * NJ + j, 0)),
        ],
        out_specs=[
            pl.BlockSpec((1, 1, 128), lambda c, j: (c * NJ + j, 0, 0)),
            pl.BlockSpec((1, 1, 128), lambda c, j: (c * NJ + j, 0, 0)),
        ],
        out_shape=[
            jax.ShapeDtypeStruct((nblk, 1, 128), jnp.float32),
            jax.ShapeDtypeStruct((nblk, 1, 128), jnp.float32),
        ],
        scratch_shapes=[pltpu.VMEM((D, B), jnp.float32)],
        compiler_params=pltpu.CompilerParams(
            dimension_semantics=("parallel", "arbitrary"),
            vmem_limit_bytes=100 * 1024 * 1024,
        ),
    )(featsT, comb_row, comb_col)

    total = jnp.sum(out_s[:, 0, 0])
    n_valid = jnp.sum(out_c[:, 0, 0])
    loss = -total / jnp.maximum(n_valid, 1.0)
    return jnp.where(n_valid > 0, loss, 0.0)


# speedup vs baseline: 1.9107x; 1.9107x over previous
import jax
import jax.numpy as jnp
from jax import lax
from jax.experimental import pallas as pl
from jax.experimental.pallas import tpu as pltpu

B = 8192
D = 256
BM = 256
BN = 256
NCORE = 2
NJ = (B // BM) // NCORE
NT = B // BN
SCALE = 4.539817985126859
LN2 = 0.6931471805599453


def _loss_kernel(featsT_ref, comb_ref, labcol_ref, out_s_ref, out_c_ref,
                 scf_ref):
    c = pl.program_id(0)
    j = pl.program_id(1)
    r = c * NJ + j

    @pl.when(j == 0)
    def _prologue():
        ft = featsT_ref[...]
        ss = jnp.sum(ft * ft, axis=0, keepdims=True)
        inv = lax.rsqrt(ss) * SCALE
        scf_ref[...] = (ft * inv).astype(jnp.bfloat16)

    lhs = scf_ref[:, pl.ds(pl.multiple_of(r * BM, BM), BM)]
    rl = labcol_ref[...][:, 0:1]
    rid = lax.broadcasted_iota(jnp.int32, (BM, 1), 0) + r * BM

    acc_e = jnp.zeros((BM, 128), jnp.float32)
    acc_p = jnp.zeros((BM, 128), jnp.float32)
    acc_c = jnp.zeros((BM, 128), jnp.float32)

    def fold(x):
        return x[:, 0:128] + x[:, 128:256]

    for jc in range(NT):
        rhs = scf_ref[:, jc * BN:(jc + 1) * BN]
        s = lax.dot_general(lhs, rhs, (((0,), (0,)), ((), ())),
                            preferred_element_type=jnp.float32)
        ct = comb_ref[0:1, jc * BN:(jc + 1) * BN]
        cid = lax.broadcasted_iota(jnp.int32, (1, BN), 1) + jc * BN
        eq = rl == ct
        dne = rid != cid
        pos = jnp.logical_and(eq, dne)
        e = jnp.where(dne, jnp.exp2(s), 0.0)
        ps = jnp.where(pos, s, 0.0)
        cs = jnp.where(pos, 1.0, 0.0)
        acc_e = acc_e + fold(e)
        acc_p = acc_p + fold(ps)
        acc_c = acc_c + fold(cs)

    se = jnp.sum(acc_e, axis=1, keepdims=True)
    lse = jnp.log(se)
    cnt = jnp.sum(acc_c, axis=1, keepdims=True)
    psum = jnp.sum(acc_p, axis=1, keepdims=True)
    mean = (psum * LN2 - cnt * lse) / (cnt + 1e-9)
    valid = cnt > 0
    contrib = jnp.where(valid, mean, 0.0)
    nv = jnp.where(valid, 1.0, 0.0)
    srow = jnp.sum(contrib, axis=0, keepdims=True)
    nrow = jnp.sum(nv, axis=0, keepdims=True)
    out_s_ref[...] = jnp.broadcast_to(srow, (1, 128)).reshape(1, 1, 128)
    out_c_ref[...] = jnp.broadcast_to(nrow, (1, 128)).reshape(1, 1, 128)


def kernel(features, concept_labels, class_labels):
    featsT = features.T
    comb = (concept_labels.astype(jnp.int32) * 16
            + class_labels.astype(jnp.int32))
    comb_row = comb.reshape(1, B)
    comb_col = jnp.broadcast_to(comb.reshape(B, 1), (B, 128))

    nblk = NCORE * NJ
    out_s, out_c = pl.pallas_call(
        _loss_kernel,
        grid=(NCORE, NJ),
        in_specs=[
            pl.BlockSpec((D, B), lambda c, j: (0, 0)),
            pl.BlockSpec((1, B), lambda c, j: (0, 0)),
            pl.BlockSpec((BM, 128), lambda c, j: (c * NJ + j, 0)),
        ],
        out_specs=[
            pl.BlockSpec((1, 1, 128), lambda c, j: (c * NJ + j, 0, 0)),
            pl.BlockSpec((1, 1, 128), lambda c, j: (c * NJ + j, 0, 0)),
        ],
        out_shape=[
            jax.ShapeDtypeStruct((nblk, 1, 128), jnp.float32),
            jax.ShapeDtypeStruct((nblk, 1, 128), jnp.float32),
        ],
        scratch_shapes=[pltpu.VMEM((D, B), jnp.bfloat16)],
        compiler_params=pltpu.CompilerParams(
            dimension_semantics=("parallel", "arbitrary"),
            vmem_limit_bytes=100 * 1024 * 1024,
        ),
    )(featsT, comb_row, comb_col)

    total = jnp.sum(out_s[:, 0, 0])
    n_valid = jnp.sum(out_c[:, 0, 0])
    loss = -total / jnp.maximum(n_valid, 1.0)
    return jnp.where(n_valid > 0, loss, 0.0)
